# baseline (speedup 1.0000x reference)
import functools

import jax
import jax.numpy as jnp
from jax.experimental import pallas as pl
from jax.experimental.pallas import tpu as pltpu

_TT = 128


def _moe_unit_kernel(meta_ref, x_ref, gate_ref, up_ref, down_ref, out_ref, *, tt):
    g = pl.program_id(0)

    @pl.when(g == 0)
    def _zero():
        out_ref[...] = jnp.zeros_like(out_ref)

    ws = pl.multiple_of(meta_ref[1, g], 8)
    rs = meta_ref[2, g]
    re = meta_ref[3, g]

    @pl.when(rs < re)
    def _compute():
        x = x_ref[pl.ds(ws, tt), :]
        gw = gate_ref[0]
        uw = up_ref[0]
        dw = down_ref[0]
        dn = (((1,), (1,)), ((), ()))
        gg = jax.lax.dot_general(x, gw, dn, preferred_element_type=jnp.float32)
        uu = jax.lax.dot_general(x, uw, dn, preferred_element_type=jnp.float32)
        act = gg * jax.nn.sigmoid(gg) * uu
        y = jax.lax.dot_general(act, dw, dn, preferred_element_type=jnp.float32)
        rows = ws + jax.lax.broadcasted_iota(jnp.int32, (tt, 1), 0)
        keep = (rows >= rs) & (rows < re)
        sl = pl.ds(ws, tt)
        out_ref[sl, :] = jnp.where(keep, y, out_ref[sl, :])


def _build_units(seg_lo, seg_hi, t_tokens, tt, n_units):
    e = seg_lo.shape[0]
    lo = jnp.clip(seg_lo, 0, t_tokens)
    hi = jnp.clip(seg_hi, 0, t_tokens)
    nonempty = hi > lo
    lo8 = (lo // 8) * 8
    nu = jnp.where(nonempty, (hi - lo8 + tt - 1) // tt, 0)
    ii = jnp.arange(e, dtype=jnp.int32)
    cum = jnp.sum(jnp.where(ii[None, :] <= ii[:, None], nu[None, :], 0), axis=1)
    total = jnp.sum(nu)
    u = jnp.arange(n_units, dtype=jnp.int32)
    eu = jnp.sum((cum[None, :] <= u[:, None]).astype(jnp.int32), axis=1)
    euc = jnp.minimum(eu, e - 1)
    oh = ii[None, :] == euc[:, None]

    def sel(v):
        return jnp.sum(jnp.where(oh, v[None, :], 0), axis=1)

    prev = sel(cum) - sel(nu)
    ws_u = sel(lo8) + (u - prev) * tt
    rs_u = jnp.maximum(sel(lo), ws_u)
    re_u = jnp.minimum(sel(hi), ws_u + tt)
    ws_c = jnp.clip(ws_u, 0, t_tokens - tt)
    valid = u < total
    last = jnp.maximum(total - 1, 0)
    ohl = (u == last) & (total > 0)
    e_pad = jnp.sum(jnp.where(ohl, euc, 0))
    w_pad = jnp.sum(jnp.where(ohl, ws_c, 0))
    return jnp.stack([
        jnp.where(valid, euc, e_pad),
        jnp.where(valid, ws_c, w_pad),
        jnp.where(valid, rs_u, 0),
        jnp.where(valid, re_u, 0),
    ])


@jax.jit
def kernel(hidden_states, experts_indices, start_indices, end_indices, gate_w, up_w, down_w):
    del experts_indices
    t_tokens, d = hidden_states.shape
    e, ff, _ = gate_w.shape
    tt = _TT
    n_units = t_tokens // tt + e + 1

    s = start_indices.astype(jnp.int32)
    seg_lo = s
    seg_hi = jnp.minimum(
        end_indices.astype(jnp.int32),
        jnp.concatenate([s[1:], jnp.full((1,), t_tokens, jnp.int32)]),
    )
    meta = _build_units(seg_lo, seg_hi, t_tokens, tt, n_units)

    grid_spec = pltpu.PrefetchScalarGridSpec(
        num_scalar_prefetch=1,
        grid=(n_units,),
        in_specs=[
            pl.BlockSpec((t_tokens, d), lambda g, m: (0, 0)),
            pl.BlockSpec((1, ff, d), lambda g, m: (m[0, g], 0, 0)),
            pl.BlockSpec((1, ff, d), lambda g, m: (m[0, g], 0, 0)),
            pl.BlockSpec((1, d, ff), lambda g, m: (m[0, g], 0, 0)),
        ],
        out_specs=pl.BlockSpec((t_tokens, d), lambda g, m: (0, 0)),
    )
    return pl.pallas_call(
        functools.partial(_moe_unit_kernel, tt=tt),
        grid_spec=grid_spec,
        out_shape=jax.ShapeDtypeStruct((t_tokens, d), jnp.float32),
        compiler_params=pltpu.CompilerParams(
            dimension_semantics=("arbitrary",),
        ),
    )(meta, hidden_states, gate_w, up_w, down_w)


# speedup vs baseline: 1.1699x; 1.1699x over previous
import functools

import jax
import jax.numpy as jnp
from jax.experimental import pallas as pl
from jax.experimental.pallas import tpu as pltpu

_TT = 256


def _moe_unit_kernel(meta_ref, x_ref, gate_ref, up_ref, down_ref, out_ref, *, tt):
    g = pl.program_id(0)

    @pl.when(g == 0)
    def _zero():
        out_ref[...] = jnp.zeros_like(out_ref)

    ws = pl.multiple_of(meta_ref[1, g], 8)
    rs = meta_ref[2, g]
    re = meta_ref[3, g]

    @pl.when(rs < re)
    def _compute():
        x = x_ref[pl.ds(ws, tt), :]
        gw = gate_ref[0]
        uw = up_ref[0]
        dw = down_ref[0]
        dn = (((1,), (1,)), ((), ()))
        gg = jax.lax.dot_general(x, gw, dn, preferred_element_type=jnp.float32)
        uu = jax.lax.dot_general(x, uw, dn, preferred_element_type=jnp.float32)
        act = gg * jax.nn.sigmoid(gg) * uu
        y = jax.lax.dot_general(act, dw, dn, preferred_element_type=jnp.float32)
        rows = ws + jax.lax.broadcasted_iota(jnp.int32, (tt, 1), 0)
        keep = (rows >= rs) & (rows < re)
        sl = pl.ds(ws, tt)
        out_ref[sl, :] = jnp.where(keep, y, out_ref[sl, :])


def _build_units(seg_lo, seg_hi, t_tokens, tt, n_units):
    e = seg_lo.shape[0]
    lo = jnp.clip(seg_lo, 0, t_tokens)
    hi = jnp.clip(seg_hi, 0, t_tokens)
    nonempty = hi > lo
    lo8 = (lo // 8) * 8
    nu = jnp.where(nonempty, (hi - lo8 + tt - 1) // tt, 0)
    ii = jnp.arange(e, dtype=jnp.int32)
    cum = jnp.sum(jnp.where(ii[None, :] <= ii[:, None], nu[None, :], 0), axis=1)
    total = jnp.sum(nu)
    u = jnp.arange(n_units, dtype=jnp.int32)
    eu = jnp.sum((cum[None, :] <= u[:, None]).astype(jnp.int32), axis=1)
    euc = jnp.minimum(eu, e - 1)
    oh = ii[None, :] == euc[:, None]

    def sel(v):
        return jnp.sum(jnp.where(oh, v[None, :], 0), axis=1)

    prev = sel(cum) - sel(nu)
    ws_u = sel(lo8) + (u - prev) * tt
    rs_u = jnp.maximum(sel(lo), ws_u)
    re_u = jnp.minimum(sel(hi), ws_u + tt)
    ws_c = jnp.clip(ws_u, 0, t_tokens - tt)
    valid = u < total
    last = jnp.maximum(total - 1, 0)
    ohl = (u == last) & (total > 0)
    e_pad = jnp.sum(jnp.where(ohl, euc, 0))
    w_pad = jnp.sum(jnp.where(ohl, ws_c, 0))
    return jnp.stack([
        jnp.where(valid, euc, e_pad),
        jnp.where(valid, ws_c, w_pad),
        jnp.where(valid, rs_u, 0),
        jnp.where(valid, re_u, 0),
    ])


@jax.jit
def kernel(hidden_states, experts_indices, start_indices, end_indices, gate_w, up_w, down_w):
    del experts_indices
    t_tokens, d = hidden_states.shape
    e, ff, _ = gate_w.shape
    tt = _TT
    n_units = t_tokens // tt + e + 1

    s = start_indices.astype(jnp.int32)
    seg_lo = s
    seg_hi = jnp.minimum(
        end_indices.astype(jnp.int32),
        jnp.concatenate([s[1:], jnp.full((1,), t_tokens, jnp.int32)]),
    )
    meta = _build_units(seg_lo, seg_hi, t_tokens, tt, n_units)

    grid_spec = pltpu.PrefetchScalarGridSpec(
        num_scalar_prefetch=1,
        grid=(n_units,),
        in_specs=[
            pl.BlockSpec((t_tokens, d), lambda g, m: (0, 0)),
            pl.BlockSpec((1, ff, d), lambda g, m: (m[0, g], 0, 0)),
            pl.BlockSpec((1, ff, d), lambda g, m: (m[0, g], 0, 0)),
            pl.BlockSpec((1, d, ff), lambda g, m: (m[0, g], 0, 0)),
        ],
        out_specs=pl.BlockSpec((t_tokens, d), lambda g, m: (0, 0)),
    )
    return pl.pallas_call(
        functools.partial(_moe_unit_kernel, tt=tt),
        grid_spec=grid_spec,
        out_shape=jax.ShapeDtypeStruct((t_tokens, d), jnp.float32),
        compiler_params=pltpu.CompilerParams(
            dimension_semantics=("arbitrary",),
        ),
    )(meta, hidden_states, gate_w, up_w, down_w)
